# baseline (speedup 1.0000x reference)
import functools

import jax
import jax.numpy as jnp
from jax import lax
from jax.experimental import pallas as pl
from jax.experimental.pallas import tpu as pltpu
from jax.experimental.pallas import tpu_sc as plsc

_NC = 2
_NS = 16
_NW = _NC * _NS
_L = 16
_RB = 14


def _tc_repack(Wt, V, D):
    H = 1 << _RB
    BLK = 2 * H

    def body(in_ref, out_ref):
        t = jnp.transpose(in_ref[...])
        out_ref[...] = jnp.concatenate(
            [t[0:H, :], t[H:BLK, :]], axis=1).astype(jnp.bfloat16)

    grid = (V + BLK - 1) // BLK
    return pl.pallas_call(
        body,
        grid=(grid,),
        in_specs=[pl.BlockSpec((D, BLK), lambda i: (0, i))],
        out_specs=pl.BlockSpec((H, 2 * D), lambda i: (i, 0)),
        out_shape=jax.ShapeDtypeStruct((grid * H, 2 * D), jnp.bfloat16),
    )(Wt)


def _sc_scores(cw, xw, nw_flat, Wc2, Wx2, B, N, D):
    bpw = B // _NW
    CH = _L
    n_chunks = bpw // CH
    NIDX = CH * N
    PR = 2 * D

    mesh = plsc.VectorSubcoreMesh(core_axis_name="c", subcore_axis_name="s")

    @functools.partial(
        pl.kernel, mesh=mesh,
        compiler_params=pltpu.CompilerParams(
            needs_layout_passes=False, use_tc_tiling_on_sc=False),
        out_type=(jax.ShapeDtypeStruct((B,), jnp.float32),
                  jax.ShapeDtypeStruct((B * N,), jnp.float32)),
        scratch_types=[
            pltpu.VMEM((CH,), jnp.int32),
            pltpu.VMEM((CH,), jnp.int32),
            pltpu.VMEM((NIDX,), jnp.int32),
            pltpu.VMEM((CH,), jnp.int32),
            pltpu.VMEM((CH,), jnp.int32),
            pltpu.VMEM((NIDX,), jnp.int32),
            pltpu.VMEM((CH, PR), jnp.bfloat16),
            pltpu.VMEM((CH, PR), jnp.bfloat16),
            pltpu.VMEM((NIDX, PR), jnp.bfloat16),
            pltpu.VMEM((CH,), jnp.float32),
            pltpu.VMEM((NIDX,), jnp.float32),
            pltpu.SemaphoreType.DMA,
        ],
    )
    def k(cw_hbm, xw_hbm, nw_hbm, Wc_hbm, Wx_hbm, pos_hbm, neg_hbm,
          cidx, xidx, nidx, cpk, xpk, npk, cbuf, xbuf, nbuf, posb, negb, sem):
        wid = lax.axis_index("s") * _NC + lax.axis_index("c")
        base = wid * bpw
        lanes = lax.iota(jnp.int32, _L)

        def load_row(buf, row, off):
            lo = plsc.unpack(buf[row, pl.ds(off, 2 * _L)],
                             format=plsc.PackFormat.INTERLEAVED)
            hi = plsc.unpack(buf[row, pl.ds(off + 2 * _L, 2 * _L)],
                             format=plsc.PackFormat.INTERLEAVED)
            return (lo[0], lo[1], hi[0], hi[1])

        def chunk_body(g, carry):
            goff = base + g * CH
            pltpu.sync_copy(cw_hbm.at[pl.ds(goff, CH)], cidx)
            pltpu.sync_copy(xw_hbm.at[pl.ds(goff, CH)], xidx)
            pltpu.sync_copy(nw_hbm.at[pl.ds(goff * N, NIDX)], nidx)

            def pack_rows(v):
                return lax.shift_left(
                    lax.shift_right_logical(v, _RB + 1), _RB) + (
                        v & ((1 << _RB) - 1))

            for i in range(CH // _L):
                s = pl.ds(i * _L, _L)
                cpk[s] = pack_rows(cidx[s])
                xpk[s] = pack_rows(xidx[s])
            for i in range(NIDX // _L):
                s = pl.ds(i * _L, _L)
                npk[s] = pack_rows(nidx[s])
            cps = [pltpu.async_copy(Wc_hbm.at[cpk], cbuf, sem),
                   pltpu.async_copy(Wx_hbm.at[xpk], xbuf, sem)]
            j = 0
            while j < NIDX:
                w = min(128, NIDX - j)
                cps.append(pltpu.async_copy(
                    Wx_hbm.at[npk.at[pl.ds(j, w)]],
                    nbuf.at[pl.ds(j, w)], sem))
                j += w
            for cp in cps:
                cp.wait()

            def half_off(v):
                return lax.shift_left(
                    lax.shift_right_logical(v, _RB) & 1, 6)

            offc_v = half_off(cidx[...])
            offx_v = half_off(xidx[...])
            offn_v = [half_off(nidx[pl.ds(i * _L, _L)])
                      for i in range(NIDX // _L)]
            pv = jnp.zeros((_L,), jnp.float32)
            nvecs = [jnp.zeros((_L,), jnp.float32) for _ in range(NIDX // _L)]
            for e in range(CH):
                c = load_row(cbuf, e, offc_v[e])
                x = load_row(xbuf, e, offx_v[e])
                acc = c[0] * x[0]
                for k2 in range(1, 4):
                    acc = acc + c[k2] * x[k2]
                pv = jnp.where(lanes == e, jnp.sum(acc), pv)
                for n in range(N):
                    r = e * N + n
                    y = load_row(nbuf, r, offn_v[r // _L][r % _L])
                    a = c[0] * y[0]
                    for k2 in range(1, 4):
                        a = a + c[k2] * y[k2]
                    nvecs[r // _L] = jnp.where(
                        lanes == (r % _L), jnp.sum(a), nvecs[r // _L])
            posb[...] = pv
            for v in range(NIDX // _L):
                negb[pl.ds(v * _L, _L)] = nvecs[v]
            pltpu.sync_copy(posb, pos_hbm.at[pl.ds(goff, CH)])
            pltpu.sync_copy(negb, neg_hbm.at[pl.ds(goff * N, NIDX)])
            return carry

        lax.fori_loop(0, n_chunks, chunk_body, 0)

    return k(cw, xw, nw_flat, Wc2, Wx2)


def _tc_loss(pos2d, neg2d, B):
    def body(pos_ref, neg_ref, out_ref):
        def ls(x):
            return jnp.minimum(x, 0.0) - jnp.log1p(jnp.exp(-jnp.abs(x)))
        tot = jnp.sum(ls(pos_ref[...])) + jnp.sum(ls(-neg_ref[...]))
        out_ref[0, 0] = -tot / B

    return pl.pallas_call(
        body,
        out_shape=jax.ShapeDtypeStruct((1, 1), jnp.float32),
        out_specs=pl.BlockSpec(memory_space=pltpu.SMEM),
    )(pos2d, neg2d)


def kernel(center_words, context_words, negative_words, W_center, W_context):
    B, N = negative_words.shape
    V, D = W_center.shape
    cw = center_words.astype(jnp.int32)
    xw = context_words.astype(jnp.int32)
    nw = negative_words.astype(jnp.int32).reshape(B * N)
    Wc2 = _tc_repack(W_center.T, V, D)
    Wx2 = _tc_repack(W_context.T, V, D)
    pos, neg = _sc_scores(cw, xw, nw, Wc2, Wx2, B, N, D)
    loss = _tc_loss(pos.reshape(B // 128, 128), neg.reshape(B * N // 128, 128), B)
    return loss[0, 0]


# speedup vs baseline: 1.0695x; 1.0695x over previous
import functools

import jax
import jax.numpy as jnp
from jax import lax
from jax.experimental import pallas as pl
from jax.experimental.pallas import tpu as pltpu
from jax.experimental.pallas import tpu_sc as plsc

_NC = 2
_NS = 16
_NW = _NC * _NS
_L = 16
_RB = 14


def _tc_repack(Wt, V, D):
    H = 1 << _RB
    BLK = 2 * H

    def body(in_ref, out_ref):
        t = jnp.transpose(in_ref[...])
        out_ref[...] = jnp.concatenate(
            [t[0:H, :], t[H:BLK, :]], axis=1).astype(jnp.bfloat16)

    grid = (V + BLK - 1) // BLK
    return pl.pallas_call(
        body,
        grid=(grid,),
        in_specs=[pl.BlockSpec((D, BLK), lambda i: (0, i))],
        out_specs=pl.BlockSpec((H, 2 * D), lambda i: (i, 0)),
        out_shape=jax.ShapeDtypeStruct((grid * H, 2 * D), jnp.bfloat16),
    )(Wt)


def _sc_scores(cw, xw, nw_flat, Wc2, Wx2, B, N, D):
    bpw = B // _NW
    CH = _L
    n_chunks = bpw // CH
    NIDX = CH * N
    PR = 2 * D

    mesh = plsc.VectorSubcoreMesh(core_axis_name="c", subcore_axis_name="s")

    @functools.partial(
        pl.kernel, mesh=mesh,
        compiler_params=pltpu.CompilerParams(
            needs_layout_passes=False, use_tc_tiling_on_sc=False),
        out_type=(jax.ShapeDtypeStruct((B,), jnp.float32),
                  jax.ShapeDtypeStruct((B * N,), jnp.float32)),
        scratch_types=[
            pltpu.VMEM((bpw,), jnp.int32),
            pltpu.VMEM((bpw,), jnp.int32),
            pltpu.VMEM((bpw * N,), jnp.int32),
            pltpu.VMEM((bpw,), jnp.int32),
            pltpu.VMEM((bpw,), jnp.int32),
            pltpu.VMEM((bpw * N,), jnp.int32),
            pltpu.VMEM((2, CH, PR), jnp.bfloat16),
            pltpu.VMEM((2, CH, PR), jnp.bfloat16),
            pltpu.VMEM((2, NIDX, PR), jnp.bfloat16),
            pltpu.VMEM((bpw,), jnp.float32),
            pltpu.VMEM((bpw * N,), jnp.float32),
            pltpu.SemaphoreType.DMA,
        ],
    )
    def k(cw_hbm, xw_hbm, nw_hbm, Wc_hbm, Wx_hbm, pos_hbm, neg_hbm,
          cidx, xidx, nidx, cpk, xpk, npk, cbuf, xbuf, nbuf, posb, negb, sem):
        wid = lax.axis_index("s") * _NC + lax.axis_index("c")
        base = wid * bpw
        lanes = lax.iota(jnp.int32, _L)

        pltpu.sync_copy(cw_hbm.at[pl.ds(base, bpw)], cidx)
        pltpu.sync_copy(xw_hbm.at[pl.ds(base, bpw)], xidx)
        pltpu.sync_copy(nw_hbm.at[pl.ds(base * N, bpw * N)], nidx)

        def pack_rows(v):
            return lax.shift_left(
                lax.shift_right_logical(v, _RB + 1), _RB) + (
                    v & ((1 << _RB) - 1))

        def pack_body(i, carry):
            s = pl.ds(i * _L, _L)
            cpk[s] = pack_rows(cidx[s])
            xpk[s] = pack_rows(xidx[s])
            return carry

        def pack_body_n(i, carry):
            s = pl.ds(i * _L, _L)
            npk[s] = pack_rows(nidx[s])
            return carry

        lax.fori_loop(0, bpw // _L, pack_body, 0)
        lax.fori_loop(0, bpw * N // _L, pack_body_n, 0)

        def fire(ch, slot):
            pltpu.async_copy(
                Wc_hbm.at[cpk.at[pl.ds(ch * CH, CH)]], cbuf.at[slot], sem)
            pltpu.async_copy(
                Wx_hbm.at[xpk.at[pl.ds(ch * CH, CH)]], xbuf.at[slot], sem)
            j = 0
            while j < NIDX:
                w = min(128, NIDX - j)
                pltpu.async_copy(
                    Wx_hbm.at[npk.at[pl.ds(ch * NIDX + j, w)]],
                    nbuf.at[slot].at[pl.ds(j, w)], sem)
                j += w

        def drain(slot):
            pltpu.make_async_copy(
                Wc_hbm.at[cpk.at[pl.ds(0, CH)]], cbuf.at[slot], sem).wait()
            pltpu.make_async_copy(
                Wx_hbm.at[xpk.at[pl.ds(0, CH)]], xbuf.at[slot], sem).wait()
            j = 0
            while j < NIDX:
                w = min(128, NIDX - j)
                pltpu.make_async_copy(
                    Wx_hbm.at[npk.at[pl.ds(j, w)]],
                    nbuf.at[slot].at[pl.ds(j, w)], sem).wait()
                j += w

        fire(0, 0)

        def half_off(v):
            return lax.shift_left(lax.shift_right_logical(v, _RB) & 1, 6)

        def load_row(buf, slot, row, off):
            lo = plsc.unpack(buf[slot, row, pl.ds(off, 2 * _L)],
                             format=plsc.PackFormat.INTERLEAVED)
            hi = plsc.unpack(buf[slot, row, pl.ds(off + 2 * _L, 2 * _L)],
                             format=plsc.PackFormat.INTERLEAVED)
            return (lo[0], lo[1], hi[0], hi[1])

        def chunk_body(g, carry):
            slot = lax.rem(g, 2)
            drain(slot)

            @pl.when(g + 1 < n_chunks)
            def _():
                fire(g + 1, 1 - slot)

            offc_v = half_off(cidx[pl.ds(g * CH, CH)])
            offx_v = half_off(xidx[pl.ds(g * CH, CH)])
            offn_v = [half_off(nidx[pl.ds(g * NIDX + i * _L, _L)])
                      for i in range(NIDX // _L)]
            pv = jnp.zeros((_L,), jnp.float32)
            nvecs = [jnp.zeros((_L,), jnp.float32) for _ in range(NIDX // _L)]
            for e in range(CH):
                c = load_row(cbuf, slot, e, offc_v[e])
                x = load_row(xbuf, slot, e, offx_v[e])
                acc = c[0] * x[0]
                for k2 in range(1, 4):
                    acc = acc + c[k2] * x[k2]
                pv = jnp.where(lanes == e, jnp.sum(acc), pv)
                for n in range(N):
                    r = e * N + n
                    y = load_row(nbuf, slot, r, offn_v[r // _L][r % _L])
                    a = c[0] * y[0]
                    for k2 in range(1, 4):
                        a = a + c[k2] * y[k2]
                    nvecs[r // _L] = jnp.where(
                        lanes == (r % _L), jnp.sum(a), nvecs[r // _L])
            posb[pl.ds(g * CH, _L)] = pv
            for v in range(NIDX // _L):
                negb[pl.ds(g * NIDX + v * _L, _L)] = nvecs[v]
            return carry

        lax.fori_loop(0, n_chunks, chunk_body, 0)
        pltpu.sync_copy(posb, pos_hbm.at[pl.ds(base, bpw)])
        pltpu.sync_copy(negb, neg_hbm.at[pl.ds(base * N, bpw * N)])

    return k(cw, xw, nw_flat, Wc2, Wx2)


def _tc_loss(pos2d, neg2d, B):
    def body(pos_ref, neg_ref, out_ref):
        def ls(x):
            return jnp.minimum(x, 0.0) - jnp.log1p(jnp.exp(-jnp.abs(x)))
        tot = jnp.sum(ls(pos_ref[...])) + jnp.sum(ls(-neg_ref[...]))
        out_ref[0, 0] = -tot / B

    return pl.pallas_call(
        body,
        out_shape=jax.ShapeDtypeStruct((1, 1), jnp.float32),
        out_specs=pl.BlockSpec(memory_space=pltpu.SMEM),
    )(pos2d, neg2d)


def kernel(center_words, context_words, negative_words, W_center, W_context):
    B, N = negative_words.shape
    V, D = W_center.shape
    cw = center_words.astype(jnp.int32)
    xw = context_words.astype(jnp.int32)
    nw = negative_words.astype(jnp.int32).reshape(B * N)
    Wc2 = _tc_repack(W_center.T, V, D)
    Wx2 = _tc_repack(W_context.T, V, D)
    pos, neg = _sc_scores(cw, xw, nw, Wc2, Wx2, B, N, D)
    loss = _tc_loss(pos.reshape(B // 128, 128), neg.reshape(B * N // 128, 128), B)
    return loss[0, 0]


# speedup vs baseline: 1.0700x; 1.0005x over previous
import functools

import jax
import jax.numpy as jnp
from jax import lax
from jax.experimental import pallas as pl
from jax.experimental.pallas import tpu as pltpu
from jax.experimental.pallas import tpu_sc as plsc

_NC = 2
_NS = 16
_NW = _NC * _NS
_L = 16
_RB = 14


def _tc_repack(Wt, V, D):
    H = 1 << _RB
    BLK = 2 * H

    def body(in_ref, out_ref):
        t = jnp.transpose(in_ref[...].astype(jnp.bfloat16))
        out_ref[...] = jnp.concatenate([t[0:H, :], t[H:BLK, :]], axis=1)

    grid = (V + BLK - 1) // BLK
    return pl.pallas_call(
        body,
        grid=(grid,),
        in_specs=[pl.BlockSpec((D, BLK), lambda i: (0, i))],
        out_specs=pl.BlockSpec((H, 2 * D), lambda i: (i, 0)),
        out_shape=jax.ShapeDtypeStruct((grid * H, 2 * D), jnp.bfloat16),
    )(Wt)


def _sc_scores(cw, xw, nw_flat, Wc2, Wx2, B, N, D):
    bpw = B // _NW
    CH = _L
    n_chunks = bpw // CH
    NIDX = CH * N
    PR = 2 * D

    mesh = plsc.VectorSubcoreMesh(core_axis_name="c", subcore_axis_name="s")

    @functools.partial(
        pl.kernel, mesh=mesh,
        compiler_params=pltpu.CompilerParams(
            needs_layout_passes=False, use_tc_tiling_on_sc=False),
        out_type=(jax.ShapeDtypeStruct((B,), jnp.float32),
                  jax.ShapeDtypeStruct((B * N,), jnp.float32)),
        scratch_types=[
            pltpu.VMEM((bpw,), jnp.int32),
            pltpu.VMEM((bpw,), jnp.int32),
            pltpu.VMEM((bpw * N,), jnp.int32),
            pltpu.VMEM((bpw,), jnp.int32),
            pltpu.VMEM((bpw,), jnp.int32),
            pltpu.VMEM((bpw * N,), jnp.int32),
            pltpu.VMEM((2, CH, PR), jnp.bfloat16),
            pltpu.VMEM((2, CH, PR), jnp.bfloat16),
            pltpu.VMEM((2, NIDX, PR), jnp.bfloat16),
            pltpu.VMEM((bpw,), jnp.float32),
            pltpu.VMEM((bpw * N,), jnp.float32),
            pltpu.SemaphoreType.DMA,
        ],
    )
    def k(cw_hbm, xw_hbm, nw_hbm, Wc_hbm, Wx_hbm, pos_hbm, neg_hbm,
          cidx, xidx, nidx, cpk, xpk, npk, cbuf, xbuf, nbuf, posb, negb, sem):
        wid = lax.axis_index("s") * _NC + lax.axis_index("c")
        base = wid * bpw
        lanes = lax.iota(jnp.int32, _L)

        pltpu.sync_copy(cw_hbm.at[pl.ds(base, bpw)], cidx)
        pltpu.sync_copy(xw_hbm.at[pl.ds(base, bpw)], xidx)
        pltpu.sync_copy(nw_hbm.at[pl.ds(base * N, bpw * N)], nidx)

        def pack_rows(v):
            return lax.shift_left(
                lax.shift_right_logical(v, _RB + 1), _RB) + (
                    v & ((1 << _RB) - 1))

        def pack_body(i, carry):
            s = pl.ds(i * _L, _L)
            cpk[s] = pack_rows(cidx[s])
            xpk[s] = pack_rows(xidx[s])
            return carry

        def pack_body_n(i, carry):
            s = pl.ds(i * _L, _L)
            npk[s] = pack_rows(nidx[s])
            return carry

        lax.fori_loop(0, bpw // _L, pack_body, 0)
        lax.fori_loop(0, bpw * N // _L, pack_body_n, 0)

        def fire(ch, slot):
            pltpu.async_copy(
                Wc_hbm.at[cpk.at[pl.ds(ch * CH, CH)]], cbuf.at[slot], sem)
            pltpu.async_copy(
                Wx_hbm.at[xpk.at[pl.ds(ch * CH, CH)]], xbuf.at[slot], sem)
            j = 0
            while j < NIDX:
                w = min(128, NIDX - j)
                pltpu.async_copy(
                    Wx_hbm.at[npk.at[pl.ds(ch * NIDX + j, w)]],
                    nbuf.at[slot].at[pl.ds(j, w)], sem)
                j += w

        def drain(slot):
            pltpu.make_async_copy(
                Wc_hbm.at[cpk.at[pl.ds(0, CH)]], cbuf.at[slot], sem).wait()
            pltpu.make_async_copy(
                Wx_hbm.at[xpk.at[pl.ds(0, CH)]], xbuf.at[slot], sem).wait()
            j = 0
            while j < NIDX:
                w = min(128, NIDX - j)
                pltpu.make_async_copy(
                    Wx_hbm.at[npk.at[pl.ds(j, w)]],
                    nbuf.at[slot].at[pl.ds(j, w)], sem).wait()
                j += w

        fire(0, 0)

        def half_off(v):
            return lax.shift_left(lax.shift_right_logical(v, _RB) & 1, 6)

        def load_row(buf, slot, row, off):
            lo = plsc.unpack(buf[slot, row, pl.ds(off, 2 * _L)],
                             format=plsc.PackFormat.INTERLEAVED)
            hi = plsc.unpack(buf[slot, row, pl.ds(off + 2 * _L, 2 * _L)],
                             format=plsc.PackFormat.INTERLEAVED)
            return (lo[0], lo[1], hi[0], hi[1])

        def chunk_body(g, carry):
            slot = lax.rem(g, 2)
            drain(slot)

            @pl.when(g + 1 < n_chunks)
            def _():
                fire(g + 1, 1 - slot)

            offc_v = half_off(cidx[pl.ds(g * CH, CH)])
            offx_v = half_off(xidx[pl.ds(g * CH, CH)])
            offn_v = [half_off(nidx[pl.ds(g * NIDX + i * _L, _L)])
                      for i in range(NIDX // _L)]
            pv = jnp.zeros((_L,), jnp.float32)
            nvecs = [jnp.zeros((_L,), jnp.float32) for _ in range(NIDX // _L)]
            for e in range(CH):
                c = load_row(cbuf, slot, e, offc_v[e])
                x = load_row(xbuf, slot, e, offx_v[e])
                acc = c[0] * x[0]
                for k2 in range(1, 4):
                    acc = acc + c[k2] * x[k2]
                pv = jnp.where(lanes == e, jnp.sum(acc), pv)
                for n in range(N):
                    r = e * N + n
                    y = load_row(nbuf, slot, r, offn_v[r // _L][r % _L])
                    a = c[0] * y[0]
                    for k2 in range(1, 4):
                        a = a + c[k2] * y[k2]
                    nvecs[r // _L] = jnp.where(
                        lanes == (r % _L), jnp.sum(a), nvecs[r // _L])
            posb[pl.ds(g * CH, _L)] = pv
            for v in range(NIDX // _L):
                negb[pl.ds(g * NIDX + v * _L, _L)] = nvecs[v]
            return carry

        lax.fori_loop(0, n_chunks, chunk_body, 0)
        pltpu.sync_copy(posb, pos_hbm.at[pl.ds(base, bpw)])
        pltpu.sync_copy(negb, neg_hbm.at[pl.ds(base * N, bpw * N)])

    return k(cw, xw, nw_flat, Wc2, Wx2)


def _tc_loss(pos2d, neg2d, B):
    def body(pos_ref, neg_ref, out_ref):
        def ls(x):
            return jnp.minimum(x, 0.0) - jnp.log1p(jnp.exp(-jnp.abs(x)))
        tot = jnp.sum(ls(pos_ref[...])) + jnp.sum(ls(-neg_ref[...]))
        out_ref[0, 0] = -tot / B

    return pl.pallas_call(
        body,
        out_shape=jax.ShapeDtypeStruct((1, 1), jnp.float32),
        out_specs=pl.BlockSpec(memory_space=pltpu.SMEM),
    )(pos2d, neg2d)


def kernel(center_words, context_words, negative_words, W_center, W_context):
    B, N = negative_words.shape
    V, D = W_center.shape
    cw = center_words.astype(jnp.int32)
    xw = context_words.astype(jnp.int32)
    nw = negative_words.astype(jnp.int32).reshape(B * N)
    Wc2 = _tc_repack(W_center.T, V, D)
    Wx2 = _tc_repack(W_context.T, V, D)
    pos, neg = _sc_scores(cw, xw, nw, Wc2, Wx2, B, N, D)
    loss = _tc_loss(pos.reshape(B // 128, 128), neg.reshape(B * N // 128, 128), B)
    return loss[0, 0]
